# Initial kernel scaffold; baseline (speedup 1.0000x reference)
import jax
import jax.numpy as jnp
from jax.experimental import pallas as pl


def kernel(x, edge_index, edge_weight, Wxz, bxz, Whz, bhz, Wxr, bxr, Whr, bhr, Wxh, bxh, Whh, bhh, Wlin, blin):
    raise NotImplementedError("write your pallas kernel here")


# speedup vs baseline: 21.3707x; 21.3707x over previous
import functools

import jax
import jax.numpy as jnp
from jax import lax
from jax.experimental import pallas as pl
from jax.experimental.pallas import tpu as pltpu
from jax.experimental.pallas import tpu_sc as plsc

NC = 2
NS = 16
NW = NC * NS
LANES = 16
CHUNK = 128
F = 128

_MESH = plsc.VectorSubcoreMesh(core_axis_name="c", subcore_axis_name="s")
_SC_PARAMS = pltpu.CompilerParams(needs_layout_passes=False)


def _make_deg_kernel(np_, ch):
    npt = np_ // NS

    @functools.partial(
        pl.kernel,
        mesh=_MESH,
        compiler_params=_SC_PARAMS,
        out_type=jax.ShapeDtypeStruct((NC, np_), jnp.float32),
        scratch_types=[
            pltpu.VMEM((ch, CHUNK), jnp.int32),
            pltpu.VMEM((ch, CHUNK), jnp.float32),
            pltpu.VMEM((npt,), jnp.float32),
            pltpu.VMEM_SHARED((np_,), jnp.float32),
        ],
    )
    def deg_kernel(src_hbm, ew_hbm, out_hbm, srcv, eww, zrow, degsh):
        c = lax.axis_index("c")
        s = lax.axis_index("s")
        wid = c * NS + s
        pltpu.sync_copy(src_hbm.at[wid], srcv)
        pltpu.sync_copy(ew_hbm.at[wid], eww)

        def zloop(i, carry):
            zrow[pl.ds(i * LANES, LANES)] = jnp.zeros((LANES,), jnp.float32)
            return carry

        lax.fori_loop(0, npt // LANES, zloop, 0)
        pltpu.sync_copy(zrow, degsh.at[pl.ds(s * npt, npt)])
        plsc.subcore_barrier()

        def chunk_loop(j, carry):
            pltpu.sync_copy(eww.at[j], degsh.at[srcv.at[j]], add=True)
            return carry

        lax.fori_loop(0, ch, chunk_loop, 0)
        plsc.subcore_barrier()
        pltpu.sync_copy(degsh.at[pl.ds(s * npt, npt)],
                        out_hbm.at[c, pl.ds(s * npt, npt)])

    return deg_kernel


def _make_norm_kernel(np_, ch):
    @functools.partial(
        pl.kernel,
        mesh=_MESH,
        compiler_params=_SC_PARAMS,
        out_type=jax.ShapeDtypeStruct((NW, ch, CHUNK), jnp.float32),
        scratch_types=[
            pltpu.VMEM((np_,), jnp.float32),
            pltpu.VMEM((ch, CHUNK), jnp.int32),
            pltpu.VMEM((ch, CHUNK), jnp.int32),
            pltpu.VMEM((ch, CHUNK), jnp.float32),
            pltpu.VMEM((ch, CHUNK), jnp.float32),
        ],
    )
    def norm_kernel(dinv_hbm, src_hbm, dst_hbm, ew_hbm, out_hbm,
                    dinvv, srcv, dstv, eww, normv):
        c = lax.axis_index("c")
        s = lax.axis_index("s")
        wid = c * NS + s
        pltpu.sync_copy(dinv_hbm, dinvv)
        pltpu.sync_copy(src_hbm.at[wid], srcv)
        pltpu.sync_copy(dst_hbm.at[wid], dstv)
        pltpu.sync_copy(ew_hbm.at[wid], eww)

        def jloop(j, carry):
            for v in range(CHUNK // LANES):
                sl = pl.ds(v * LANES, LANES)
                i16 = srcv[j, sl]
                d16 = dstv[j, sl]
                w16 = eww[j, sl]
                gs = plsc.load_gather(dinvv, [i16])
                gd = plsc.load_gather(dinvv, [d16])
                normv[j, sl] = -(gs * w16 * gd)
            return carry

        lax.fori_loop(0, ch, jloop, 0)
        pltpu.sync_copy(normv, out_hbm.at[wid])

    return norm_kernel


def _make_spmm_kernel(np_, ch):
    npt = np_ // NS

    @functools.partial(
        pl.kernel,
        mesh=_MESH,
        compiler_params=_SC_PARAMS,
        out_type=jax.ShapeDtypeStruct((NC, np_, F), jnp.float32),
        scratch_types=[
            pltpu.VMEM((ch, CHUNK), jnp.int32),
            pltpu.VMEM((ch, CHUNK), jnp.int32),
            pltpu.VMEM((ch, CHUNK), jnp.float32),
            pltpu.VMEM((CHUNK, F), jnp.float32),
            pltpu.VMEM_SHARED((np_, F), jnp.float32),
            pltpu.SemaphoreType.DMA,
        ],
    )
    def spmm_kernel(x_hbm, src_hbm, dst_hbm, nrm_hbm, out_hbm,
                    srcv, dstv, normv, rows, accsh, sem):
        c = lax.axis_index("c")
        s = lax.axis_index("s")
        wid = c * NS + s
        pltpu.sync_copy(src_hbm.at[wid], srcv)
        pltpu.sync_copy(dst_hbm.at[wid], dstv)
        pltpu.sync_copy(nrm_hbm.at[wid], normv)

        def zloop(i, carry):
            for v in range(F // LANES):
                rows[i, pl.ds(v * LANES, LANES)] = jnp.zeros((LANES,),
                                                             jnp.float32)
            return carry

        lax.fori_loop(0, CHUNK, zloop, 0)
        for k in range(npt // CHUNK):
            pltpu.sync_copy(rows,
                            accsh.at[pl.ds(s * npt + k * CHUNK, CHUNK)])
        plsc.subcore_barrier()

        def chunk_loop(j, carry):
            pltpu.async_copy(x_hbm.at[srcv.at[j]], rows, sem).wait()

            def scale_loop(g, inner):
                nv = normv[j, pl.ds(g * LANES, LANES)]
                base = g * LANES
                for r in range(LANES):
                    sc = nv[r]
                    for v in range(F // LANES):
                        sl = pl.ds(v * LANES, LANES)
                        rows[base + r, sl] = rows[base + r, sl] * sc
                return inner

            lax.fori_loop(0, CHUNK // LANES, scale_loop, 0)
            pltpu.sync_copy(rows, accsh.at[dstv.at[j]], add=True)
            return carry

        lax.fori_loop(0, ch, chunk_loop, 0)
        plsc.subcore_barrier()
        pltpu.sync_copy(accsh.at[pl.ds(s * npt, npt)],
                        out_hbm.at[c, pl.ds(s * npt, npt)])

    return spmm_kernel


def _dinv_body(d2_ref, o_ref):
    d = d2_ref[0] + d2_ref[1]
    safe = jnp.where(d > 0, d, 1.0)
    o_ref[...] = jnp.where(d > 0, 1.0 / jnp.sqrt(safe), 0.0)


def _comb_add_body(pa_ref, pb_ref, o_ref):
    o_ref[...] = pa_ref[...] + pb_ref[...]


def _comb_aff_body(pa_ref, pb_ref, prev_ref, o_ref):
    o_ref[...] = 2.0 * (pa_ref[...] + pb_ref[...]) - prev_ref[...]


def _final_body(t0_ref, t1_ref, t2_ref, p3a_ref, p3b_ref, wz_ref, wh_ref,
                bz_ref, bh_ref, wl_ref, bl_ref, o_ref):
    x0 = t0_ref[...]
    x1 = t1_ref[...]
    x2 = t2_ref[...]
    x3 = 2.0 * (p3a_ref[...] + p3b_ref[...]) - x1

    def conv(w_ref, b_ref):
        acc = jnp.dot(x0, w_ref[0], preferred_element_type=jnp.float32)
        acc += jnp.dot(x1, w_ref[1], preferred_element_type=jnp.float32)
        acc += jnp.dot(x2, w_ref[2], preferred_element_type=jnp.float32)
        acc += jnp.dot(x3, w_ref[3], preferred_element_type=jnp.float32)
        return acc + b_ref[...]

    az = conv(wz_ref, bz_ref)
    ah = conv(wh_ref, bh_ref)
    z = jax.nn.sigmoid(az)
    ht = jnp.tanh(ah)
    y = (1.0 - z) * ht
    o_ref[...] = (jnp.dot(y, wl_ref[...], preferred_element_type=jnp.float32)
                  + bl_ref[...])


def kernel(x, edge_index, edge_weight, Wxz, bxz, Whz, bhz, Wxr, bxr, Whr,
           bhr, Wxh, bxh, Whh, bhh, Wlin, blin):
    n, f = x.shape
    e = edge_weight.shape[0]
    np_ = ((n + 2047) // 2048) * 2048
    ch = (e + NW * CHUNK - 1) // (NW * CHUNK)
    e_pad = NW * ch * CHUNK

    src = edge_index[0].astype(jnp.int32)
    dst = edge_index[1].astype(jnp.int32)
    ew = edge_weight.astype(jnp.float32)
    pad = e_pad - e
    pad_idx = jnp.arange(pad, dtype=jnp.int32) % jnp.int32(n)
    src3 = jnp.concatenate([src, pad_idx]).reshape(NW, ch, CHUNK)
    dst3 = jnp.concatenate([dst, pad_idx]).reshape(NW, ch, CHUNK)
    ew3 = jnp.concatenate([ew, jnp.zeros((pad,), jnp.float32)]
                          ).reshape(NW, ch, CHUNK)
    xp = jnp.pad(x, ((0, np_ - n), (0, 0)))

    deg_k = _make_deg_kernel(np_, ch)
    norm_k = _make_norm_kernel(np_, ch)
    spmm_k = _make_spmm_kernel(np_, ch)

    deg2 = deg_k(src3, ew3)

    npr = np_ // 128
    dinv = pl.pallas_call(
        _dinv_body,
        out_shape=jax.ShapeDtypeStruct((npr, 128), jnp.float32),
    )(deg2.reshape(2, npr, 128)).reshape(np_)

    norm3 = norm_k(dinv, src3, dst3, ew3)

    rows_blk = 1280
    grid = (np_ // rows_blk,)
    vspec = pl.BlockSpec((rows_blk, F), lambda i: (i, 0))

    def comb_add(pa, pb):
        return pl.pallas_call(
            _comb_add_body,
            grid=grid,
            in_specs=[vspec, vspec],
            out_specs=vspec,
            out_shape=jax.ShapeDtypeStruct((np_, F), jnp.float32),
        )(pa, pb)

    def comb_aff(pa, pb, prev):
        return pl.pallas_call(
            _comb_aff_body,
            grid=grid,
            in_specs=[vspec, vspec, vspec],
            out_specs=vspec,
            out_shape=jax.ShapeDtypeStruct((np_, F), jnp.float32),
        )(pa, pb, prev)

    p1 = spmm_k(xp, src3, dst3, norm3)
    t1 = comb_add(p1[0], p1[1])
    p2 = spmm_k(t1, src3, dst3, norm3)
    t2 = comb_aff(p2[0], p2[1], xp)
    p3 = spmm_k(t2, src3, dst3, norm3)

    wl_pad = jnp.pad(Wlin, ((0, 0), (0, F - Wlin.shape[1])))
    bl_pad = jnp.pad(blin, (0, F - blin.shape[0])).reshape(1, F)
    wspec = pl.BlockSpec((4, F, F), lambda i: (0, 0, 0))
    bspec = pl.BlockSpec((1, F), lambda i: (0, 0))
    mspec = pl.BlockSpec((F, F), lambda i: (0, 0))

    out_pad = pl.pallas_call(
        _final_body,
        grid=grid,
        in_specs=[vspec, vspec, vspec, vspec, vspec,
                  wspec, wspec, bspec, bspec, mspec, bspec],
        out_specs=vspec,
        out_shape=jax.ShapeDtypeStruct((np_, F), jnp.float32),
    )(xp, t1, t2, p3[0], p3[1], Wxz, Wxh,
      (bxz + bhz).reshape(1, F), (bxh + bhh).reshape(1, F), wl_pad, bl_pad)

    return out_pad[:n, :1]


# speedup vs baseline: 29.7929x; 1.3941x over previous
import functools

import jax
import jax.numpy as jnp
from jax import lax
from jax.experimental import pallas as pl
from jax.experimental.pallas import tpu as pltpu
from jax.experimental.pallas import tpu_sc as plsc

NC = 2
NS = 16
NW = NC * NS
LANES = 16
CHUNK = 128
F = 128

_MESH = plsc.VectorSubcoreMesh(core_axis_name="c", subcore_axis_name="s")
_SC_PARAMS = pltpu.CompilerParams(needs_layout_passes=False,
                                  use_tc_tiling_on_sc=False)


def _make_deg_kernel(np_, ch):
    npt = np_ // NS

    @functools.partial(
        pl.kernel,
        mesh=_MESH,
        compiler_params=_SC_PARAMS,
        out_type=jax.ShapeDtypeStruct((NC, np_), jnp.float32),
        scratch_types=[
            pltpu.VMEM((ch, CHUNK), jnp.int32),
            pltpu.VMEM((ch, CHUNK), jnp.float32),
            pltpu.VMEM((npt,), jnp.float32),
            pltpu.VMEM_SHARED((np_,), jnp.float32),
        ],
    )
    def deg_kernel(src_hbm, ew_hbm, out_hbm, srcv, eww, zrow, degsh):
        c = lax.axis_index("c")
        s = lax.axis_index("s")
        wid = c * NS + s
        pltpu.sync_copy(src_hbm.at[wid], srcv)
        pltpu.sync_copy(ew_hbm.at[wid], eww)

        def zloop(i, carry):
            zrow[pl.ds(i * LANES, LANES)] = jnp.zeros((LANES,), jnp.float32)
            return carry

        lax.fori_loop(0, npt // LANES, zloop, 0)
        pltpu.sync_copy(zrow, degsh.at[pl.ds(s * npt, npt)])
        plsc.subcore_barrier()

        def chunk_loop(j, carry):
            pltpu.sync_copy(eww.at[j], degsh.at[srcv.at[j]], add=True)
            return carry

        lax.fori_loop(0, ch, chunk_loop, 0)
        plsc.subcore_barrier()
        pltpu.sync_copy(degsh.at[pl.ds(s * npt, npt)],
                        out_hbm.at[c, pl.ds(s * npt, npt)])

    return deg_kernel


def _make_norm_kernel(np_, ch):
    @functools.partial(
        pl.kernel,
        mesh=_MESH,
        compiler_params=_SC_PARAMS,
        out_type=jax.ShapeDtypeStruct((NW, ch, CHUNK), jnp.float32),
        scratch_types=[
            pltpu.VMEM((np_,), jnp.float32),
            pltpu.VMEM((ch, CHUNK), jnp.int32),
            pltpu.VMEM((ch, CHUNK), jnp.int32),
            pltpu.VMEM((ch, CHUNK), jnp.float32),
            pltpu.VMEM((ch, CHUNK), jnp.float32),
        ],
    )
    def norm_kernel(dinv_hbm, src_hbm, dst_hbm, ew_hbm, out_hbm,
                    dinvv, srcv, dstv, eww, normv):
        c = lax.axis_index("c")
        s = lax.axis_index("s")
        wid = c * NS + s
        pltpu.sync_copy(dinv_hbm, dinvv)
        pltpu.sync_copy(src_hbm.at[wid], srcv)
        pltpu.sync_copy(dst_hbm.at[wid], dstv)
        pltpu.sync_copy(ew_hbm.at[wid], eww)

        def jloop(j, carry):
            for v in range(CHUNK // LANES):
                sl = pl.ds(v * LANES, LANES)
                i16 = srcv[j, sl]
                d16 = dstv[j, sl]
                w16 = eww[j, sl]
                gs = plsc.load_gather(dinvv, [i16])
                gd = plsc.load_gather(dinvv, [d16])
                normv[j, sl] = -(gs * w16 * gd)
            return carry

        lax.fori_loop(0, ch, jloop, 0)
        pltpu.sync_copy(normv, out_hbm.at[wid])

    return norm_kernel


NBUF = 3
FH = 64


def _make_spmm_kernel(np_, ch):
    npt = np_ // NS

    @functools.partial(
        pl.kernel,
        mesh=_MESH,
        compiler_params=_SC_PARAMS,
        out_type=jax.ShapeDtypeStruct((NC, np_, FH), jnp.float32),
        scratch_types=[
            pltpu.VMEM((ch, CHUNK), jnp.int32),
            pltpu.VMEM((ch, CHUNK), jnp.int32),
            pltpu.VMEM((ch, CHUNK), jnp.float32),
            pltpu.VMEM_SHARED((np_, FH), jnp.float32),
            [pltpu.VMEM((CHUNK, FH), jnp.float32) for _ in range(NBUF)],
            [pltpu.VMEM((CHUNK, FH), jnp.float32) for _ in range(NBUF)],
            [pltpu.SemaphoreType.DMA for _ in range(NBUF)],
            [pltpu.SemaphoreType.DMA for _ in range(NBUF)],
        ],
    )
    def spmm_kernel(x_hbm, src_hbm, dst_hbm, nrm_hbm, out_hbm,
                    srcv, dstv, normv, accsh, grows, srows, gsem, ssem):
        c = lax.axis_index("c")
        s = lax.axis_index("s")
        wid = c * NS + s
        pltpu.sync_copy(src_hbm.at[wid], srcv)
        pltpu.sync_copy(dst_hbm.at[wid], dstv)
        pltpu.sync_copy(nrm_hbm.at[wid], normv)

        def zloop(i, carry):
            for v in range(FH // LANES):
                srows[0][i, pl.ds(v * LANES, LANES)] = jnp.zeros(
                    (LANES,), jnp.float32)
            return carry

        lax.fori_loop(0, CHUNK, zloop, 0)
        for k in range(npt // CHUNK):
            pltpu.sync_copy(srows[0],
                            accsh.at[pl.ds(s * npt + k * CHUNK, CHUNK)])
        plsc.subcore_barrier()

        for b in range(NBUF):
            pltpu.async_copy(x_hbm.at[srcv.at[b]], grows[b], gsem[b])
        for b in range(NBUF):
            pltpu.async_copy(srows[0], accsh.at[dstv.at[0]], ssem[b],
                             add=True)

        def round_loop(jj, carry):
            for b in range(NBUF):
                j = jj * NBUF + b
                pltpu.make_async_copy(x_hbm.at[srcv.at[j]], grows[b],
                                      gsem[b]).wait()
                pltpu.make_async_copy(srows[b], accsh.at[dstv.at[j]],
                                      ssem[b]).wait()

                def scale_loop(g, inner, j=j, b=b):
                    nv = normv[j, pl.ds(g * LANES, LANES)]
                    base = g * LANES
                    for r in range(LANES):
                        sc = nv[r]
                        for v in range(FH // LANES):
                            sl = pl.ds(v * LANES, LANES)
                            srows[b][base + r, sl] = (
                                grows[b][base + r, sl] * sc)
                    return inner

                lax.fori_loop(0, CHUNK // LANES, scale_loop, 0)
                pltpu.async_copy(srows[b], accsh.at[dstv.at[j]], ssem[b],
                                 add=True)
                pj = jnp.minimum(j + NBUF, ch - 1)
                pltpu.async_copy(x_hbm.at[srcv.at[pj]], grows[b], gsem[b])
            return carry

        lax.fori_loop(0, ch // NBUF, round_loop, 0)
        for b in range(NBUF):
            pltpu.make_async_copy(x_hbm.at[srcv.at[0]], grows[b],
                                  gsem[b]).wait()
            pltpu.make_async_copy(srows[b], accsh.at[dstv.at[0]],
                                  ssem[b]).wait()
        plsc.subcore_barrier()
        pltpu.sync_copy(accsh.at[pl.ds(s * npt, npt)],
                        out_hbm.at[c, pl.ds(s * npt, npt)])

    return spmm_kernel


def _dinv_body(d2_ref, o_ref):
    d = d2_ref[0] + d2_ref[1]
    safe = jnp.where(d > 0, d, 1.0)
    o_ref[...] = jnp.where(d > 0, 1.0 / jnp.sqrt(safe), 0.0)


def _comb_add_body(pa_ref, pb_ref, oa_ref, ob_ref):
    oa_ref[...] = pa_ref[0] + pa_ref[1]
    ob_ref[...] = pb_ref[0] + pb_ref[1]


def _comb_aff_body(pa_ref, pb_ref, preva_ref, prevb_ref, oa_ref, ob_ref):
    oa_ref[...] = 2.0 * (pa_ref[0] + pa_ref[1]) - preva_ref[...]
    ob_ref[...] = 2.0 * (pb_ref[0] + pb_ref[1]) - prevb_ref[...]


def _final_body(t0_ref, t1a_ref, t1b_ref, t2a_ref, t2b_ref, t3a_ref,
                t3b_ref, wz_ref, wh_ref, bz_ref, bh_ref, wl_ref, bl_ref,
                o_ref):
    x0 = t0_ref[...]
    x1 = jnp.concatenate([t1a_ref[...], t1b_ref[...]], axis=1)
    x2 = jnp.concatenate([t2a_ref[...], t2b_ref[...]], axis=1)
    x3 = jnp.concatenate([t3a_ref[...], t3b_ref[...]], axis=1)

    def conv(w_ref, b_ref):
        acc = jnp.dot(x0, w_ref[0], preferred_element_type=jnp.float32)
        acc += jnp.dot(x1, w_ref[1], preferred_element_type=jnp.float32)
        acc += jnp.dot(x2, w_ref[2], preferred_element_type=jnp.float32)
        acc += jnp.dot(x3, w_ref[3], preferred_element_type=jnp.float32)
        return acc + b_ref[...]

    az = conv(wz_ref, bz_ref)
    ah = conv(wh_ref, bh_ref)
    z = jax.nn.sigmoid(az)
    ht = jnp.tanh(ah)
    y = (1.0 - z) * ht
    o_ref[...] = (jnp.dot(y, wl_ref[...], preferred_element_type=jnp.float32)
                  + bl_ref[...])


def kernel(x, edge_index, edge_weight, Wxz, bxz, Whz, bhz, Wxr, bxr, Whr,
           bhr, Wxh, bxh, Whh, bhh, Wlin, blin):
    n, f = x.shape
    e = edge_weight.shape[0]
    np_ = ((n + 2047) // 2048) * 2048
    ch = (e + NW * CHUNK - 1) // (NW * CHUNK)
    ch = ((ch + NBUF - 1) // NBUF) * NBUF
    e_pad = NW * ch * CHUNK

    src = edge_index[0].astype(jnp.int32)
    dst = edge_index[1].astype(jnp.int32)
    ew = edge_weight.astype(jnp.float32)
    pad = e_pad - e
    pad_idx = jnp.arange(pad, dtype=jnp.int32) % jnp.int32(n)
    src3 = jnp.concatenate([src, pad_idx]).reshape(NW, ch, CHUNK)
    dst3 = jnp.concatenate([dst, pad_idx]).reshape(NW, ch, CHUNK)
    ew3 = jnp.concatenate([ew, jnp.zeros((pad,), jnp.float32)]
                          ).reshape(NW, ch, CHUNK)
    xp = jnp.pad(x, ((0, np_ - n), (0, 0)))

    deg_k = _make_deg_kernel(np_, ch)
    norm_k = _make_norm_kernel(np_, ch)
    spmm_k = _make_spmm_kernel(np_, ch)

    deg2 = deg_k(src3, ew3)

    npr = np_ // 128
    dinv = pl.pallas_call(
        _dinv_body,
        out_shape=jax.ShapeDtypeStruct((npr, 128), jnp.float32),
    )(deg2.reshape(2, npr, 128)).reshape(np_)

    norm3 = norm_k(dinv, src3, dst3, ew3)

    rows_blk = 1280
    grid = (np_ // rows_blk,)
    vspec = pl.BlockSpec((rows_blk, F), lambda i: (i, 0))
    hspec = pl.BlockSpec((rows_blk, FH), lambda i: (i, 0))
    pspec = pl.BlockSpec((2, rows_blk, FH), lambda i: (0, i, 0))
    hshape = jax.ShapeDtypeStruct((np_, FH), jnp.float32)

    def comb_add(pa, pb):
        return pl.pallas_call(
            _comb_add_body,
            grid=grid,
            in_specs=[pspec, pspec],
            out_specs=[hspec, hspec],
            out_shape=[hshape, hshape],
        )(pa, pb)

    def comb_aff(pa, pb, preva, prevb):
        return pl.pallas_call(
            _comb_aff_body,
            grid=grid,
            in_specs=[pspec, pspec, hspec, hspec],
            out_specs=[hspec, hspec],
            out_shape=[hshape, hshape],
        )(pa, pb, preva, prevb)

    xa = xp[:, :FH]
    xb = xp[:, FH:]
    p1a = spmm_k(xa, src3, dst3, norm3)
    p1b = spmm_k(xb, src3, dst3, norm3)
    t1a, t1b = comb_add(p1a, p1b)
    p2a = spmm_k(t1a, src3, dst3, norm3)
    p2b = spmm_k(t1b, src3, dst3, norm3)
    t2a, t2b = comb_aff(p2a, p2b, xa, xb)
    p3a = spmm_k(t2a, src3, dst3, norm3)
    p3b = spmm_k(t2b, src3, dst3, norm3)
    t3a, t3b = comb_aff(p3a, p3b, t1a, t1b)

    wl_pad = jnp.pad(Wlin, ((0, 0), (0, F - Wlin.shape[1])))
    bl_pad = jnp.pad(blin, (0, F - blin.shape[0])).reshape(1, F)
    wspec = pl.BlockSpec((4, F, F), lambda i: (0, 0, 0))
    bspec = pl.BlockSpec((1, F), lambda i: (0, 0))
    mspec = pl.BlockSpec((F, F), lambda i: (0, 0))

    out_pad = pl.pallas_call(
        _final_body,
        grid=grid,
        in_specs=[vspec, hspec, hspec, hspec, hspec, hspec, hspec,
                  wspec, wspec, bspec, bspec, mspec, bspec],
        out_specs=vspec,
        out_shape=jax.ShapeDtypeStruct((np_, F), jnp.float32),
    )(xp, t1a, t1b, t2a, t2b, t3a, t3b, Wxz, Wxh,
      (bxz + bhz).reshape(1, F), (bxh + bhh).reshape(1, F), wl_pad, bl_pad)

    return out_pad[:n, :1]


# speedup vs baseline: 30.4503x; 1.0221x over previous
import functools

import jax
import jax.numpy as jnp
from jax import lax
from jax.experimental import pallas as pl
from jax.experimental.pallas import tpu as pltpu
from jax.experimental.pallas import tpu_sc as plsc

NC = 2
NS = 16
NW = NC * NS
LANES = 16
CHUNK = 128
F = 128

_MESH = plsc.VectorSubcoreMesh(core_axis_name="c", subcore_axis_name="s")
_SC_PARAMS = pltpu.CompilerParams(needs_layout_passes=False,
                                  use_tc_tiling_on_sc=False)


def _rsqrt16(d):
    i = plsc.bitcast(d, jnp.int32)
    y = plsc.bitcast(jnp.int32(0x5F3759DF) - (i >> 1), jnp.float32)
    for _ in range(3):
        y = y * (1.5 - 0.5 * d * y * y)
    return jnp.where(d > 0, y, 0.0)


def _make_prep_kernel(np_, ch):
    npt = np_ // NS

    @functools.partial(
        pl.kernel,
        mesh=_MESH,
        compiler_params=_SC_PARAMS,
        out_type=jax.ShapeDtypeStruct((NW, ch, CHUNK), jnp.float32),
        scratch_types=[
            pltpu.VMEM((ch, CHUNK), jnp.int32),
            pltpu.VMEM((ch, CHUNK), jnp.float32),
            pltpu.VMEM((ch, CHUNK), jnp.int32),
            pltpu.VMEM((ch, CHUNK), jnp.float32),
            pltpu.VMEM((ch, CHUNK), jnp.int32),
            pltpu.VMEM((ch, CHUNK), jnp.float32),
            pltpu.VMEM((np_,), jnp.float32),
            pltpu.VMEM((npt,), jnp.float32),
            pltpu.VMEM_SHARED((np_,), jnp.float32),
            pltpu.SemaphoreType.DMA,
        ],
    )
    def prep_kernel(src_hbm, dst_hbm, ew_hbm, out_hbm,
                    srcA, ewA, srcB, ewB, dstv, normv, dinvv, dv, degsh,
                    sem):
        c = lax.axis_index("c")
        s = lax.axis_index("s")
        wid = c * NS + s
        pltpu.sync_copy(src_hbm.at[s], srcA)
        pltpu.sync_copy(ew_hbm.at[s], ewA)
        pltpu.sync_copy(src_hbm.at[NS + s], srcB)
        pltpu.sync_copy(ew_hbm.at[NS + s], ewB)
        pltpu.sync_copy(dst_hbm.at[wid], dstv)

        def zloop(i, carry):
            dv[pl.ds(i * LANES, LANES)] = jnp.zeros((LANES,), jnp.float32)
            return carry

        lax.fori_loop(0, npt // LANES, zloop, 0)
        pltpu.sync_copy(dv, degsh.at[pl.ds(s * npt, npt)])
        plsc.subcore_barrier()

        def deg_loop(j, carry):
            pltpu.async_copy(ewA.at[j], degsh.at[srcA.at[j]], sem,
                             add=True)
            pltpu.async_copy(ewB.at[j], degsh.at[srcB.at[j]], sem,
                             add=True)
            return carry

        lax.fori_loop(0, ch, deg_loop, 0)

        def deg_drain(j, carry):
            pltpu.make_async_copy(ewA.at[j], degsh.at[srcA.at[j]],
                                  sem).wait()
            pltpu.make_async_copy(ewB.at[j], degsh.at[srcB.at[j]],
                                  sem).wait()
            return carry

        lax.fori_loop(0, ch, deg_drain, 0)
        plsc.subcore_barrier()

        pltpu.sync_copy(degsh.at[pl.ds(s * npt, npt)], dv)

        def rloop(i, carry):
            sl = pl.ds(i * LANES, LANES)
            dv[sl] = _rsqrt16(dv[sl])
            return carry

        lax.fori_loop(0, npt // LANES, rloop, 0)
        pltpu.sync_copy(dv, degsh.at[pl.ds(s * npt, npt)])
        plsc.subcore_barrier()
        pltpu.sync_copy(degsh, dinvv)

        def jloop(j, carry, srcv, eww):
            for v in range(CHUNK // LANES):
                sl = pl.ds(v * LANES, LANES)
                i16 = srcv[j, sl]
                d16 = dstv[j, sl]
                w16 = eww[j, sl]
                gs = plsc.load_gather(dinvv, [i16])
                gd = plsc.load_gather(dinvv, [d16])
                normv[j, sl] = -(gs * w16 * gd)
            return carry

        @pl.when(c == 0)
        def _():
            lax.fori_loop(0, ch,
                          functools.partial(jloop, srcv=srcA, eww=ewA), 0)

        @pl.when(c != 0)
        def _():
            lax.fori_loop(0, ch,
                          functools.partial(jloop, srcv=srcB, eww=ewB), 0)

        pltpu.sync_copy(normv, out_hbm.at[wid])

    return prep_kernel


NBUF = 3
FH = 64


def _make_spmm_kernel(np_, ch):
    npt = np_ // NS

    hshape = jax.ShapeDtypeStruct((NC, np_, FH), jnp.float32)

    @functools.partial(
        pl.kernel,
        mesh=_MESH,
        compiler_params=_SC_PARAMS,
        out_type=(hshape, hshape),
        scratch_types=[
            pltpu.VMEM((ch, CHUNK), jnp.int32),
            pltpu.VMEM((ch, CHUNK), jnp.int32),
            pltpu.VMEM((ch, CHUNK), jnp.float32),
            pltpu.VMEM_SHARED((np_, FH), jnp.float32),
            [pltpu.VMEM((CHUNK, FH), jnp.float32) for _ in range(NBUF)],
            [pltpu.VMEM((CHUNK, FH), jnp.float32) for _ in range(NBUF)],
            [pltpu.SemaphoreType.DMA for _ in range(NBUF)],
            [pltpu.SemaphoreType.DMA for _ in range(NBUF)],
        ],
    )
    def spmm_kernel(xa_hbm, xb_hbm, src_hbm, dst_hbm, nrm_hbm,
                    outa_hbm, outb_hbm,
                    srcv, dstv, normv, accsh, grows, srows, gsem, ssem):
        c = lax.axis_index("c")
        s = lax.axis_index("s")
        wid = c * NS + s
        pltpu.sync_copy(src_hbm.at[wid], srcv)
        pltpu.sync_copy(dst_hbm.at[wid], dstv)
        pltpu.sync_copy(nrm_hbm.at[wid], normv)

        for x_hbm, out_hbm in ((xa_hbm, outa_hbm), (xb_hbm, outb_hbm)):
            def zloop(i, carry):
                for v in range(FH // LANES):
                    srows[0][i, pl.ds(v * LANES, LANES)] = jnp.zeros(
                        (LANES,), jnp.float32)
                return carry

            lax.fori_loop(0, CHUNK, zloop, 0)
            for k in range(npt // CHUNK):
                pltpu.sync_copy(srows[0],
                                accsh.at[pl.ds(s * npt + k * CHUNK, CHUNK)])
            plsc.subcore_barrier()

            for b in range(NBUF):
                pltpu.async_copy(x_hbm.at[srcv.at[b]], grows[b], gsem[b])
            for b in range(NBUF):
                pltpu.async_copy(srows[0], accsh.at[dstv.at[0]], ssem[b],
                                 add=True)

            def round_loop(jj, carry, x_hbm=x_hbm):
                for b in range(NBUF):
                    j = jj * NBUF + b
                    pltpu.make_async_copy(x_hbm.at[srcv.at[j]], grows[b],
                                          gsem[b]).wait()
                    pltpu.make_async_copy(srows[b], accsh.at[dstv.at[j]],
                                          ssem[b]).wait()

                    def scale_loop(g, inner, j=j, b=b):
                        nv = normv[j, pl.ds(g * LANES, LANES)]
                        base = g * LANES
                        for r in range(LANES):
                            sc = nv[r]
                            for v in range(FH // LANES):
                                sl = pl.ds(v * LANES, LANES)
                                srows[b][base + r, sl] = (
                                    grows[b][base + r, sl] * sc)
                        return inner

                    lax.fori_loop(0, CHUNK // LANES, scale_loop, 0)
                    pltpu.async_copy(srows[b], accsh.at[dstv.at[j]],
                                     ssem[b], add=True)
                    pj = jnp.minimum(j + NBUF, ch - 1)
                    pltpu.async_copy(x_hbm.at[srcv.at[pj]], grows[b],
                                     gsem[b])
                return carry

            lax.fori_loop(0, ch // NBUF, round_loop, 0)
            for b in range(NBUF):
                pltpu.make_async_copy(x_hbm.at[srcv.at[0]], grows[b],
                                      gsem[b]).wait()
                pltpu.make_async_copy(srows[b], accsh.at[dstv.at[0]],
                                      ssem[b]).wait()
            plsc.subcore_barrier()
            pltpu.sync_copy(accsh.at[pl.ds(s * npt, npt)],
                            out_hbm.at[c, pl.ds(s * npt, npt)])

    return spmm_kernel


def _comb_add_body(pa_ref, pb_ref, oa_ref, ob_ref):
    oa_ref[...] = pa_ref[0] + pa_ref[1]
    ob_ref[...] = pb_ref[0] + pb_ref[1]


def _comb_aff_body(pa_ref, pb_ref, preva_ref, prevb_ref, oa_ref, ob_ref):
    oa_ref[...] = 2.0 * (pa_ref[0] + pa_ref[1]) - preva_ref[...]
    ob_ref[...] = 2.0 * (pb_ref[0] + pb_ref[1]) - prevb_ref[...]


def _final_body(t0_ref, t1a_ref, t1b_ref, t2a_ref, t2b_ref, t3a_ref,
                t3b_ref, wz_ref, wh_ref, bz_ref, bh_ref, wl_ref, bl_ref,
                o_ref):
    x0 = t0_ref[...]
    x1 = jnp.concatenate([t1a_ref[...], t1b_ref[...]], axis=1)
    x2 = jnp.concatenate([t2a_ref[...], t2b_ref[...]], axis=1)
    x3 = jnp.concatenate([t3a_ref[...], t3b_ref[...]], axis=1)

    def conv(w_ref, b_ref):
        acc = jnp.dot(x0, w_ref[0], preferred_element_type=jnp.float32)
        acc += jnp.dot(x1, w_ref[1], preferred_element_type=jnp.float32)
        acc += jnp.dot(x2, w_ref[2], preferred_element_type=jnp.float32)
        acc += jnp.dot(x3, w_ref[3], preferred_element_type=jnp.float32)
        return acc + b_ref[...]

    az = conv(wz_ref, bz_ref)
    ah = conv(wh_ref, bh_ref)
    z = jax.nn.sigmoid(az)
    ht = jnp.tanh(ah)
    y = (1.0 - z) * ht
    o_ref[...] = (jnp.dot(y, wl_ref[...], preferred_element_type=jnp.float32)
                  + bl_ref[...])


def kernel(x, edge_index, edge_weight, Wxz, bxz, Whz, bhz, Wxr, bxr, Whr,
           bhr, Wxh, bxh, Whh, bhh, Wlin, blin):
    n, f = x.shape
    e = edge_weight.shape[0]
    np_ = ((n + 2047) // 2048) * 2048
    ch = (e + NW * CHUNK - 1) // (NW * CHUNK)
    ch = ((ch + NBUF - 1) // NBUF) * NBUF
    e_pad = NW * ch * CHUNK

    src = edge_index[0].astype(jnp.int32)
    dst = edge_index[1].astype(jnp.int32)
    ew = edge_weight.astype(jnp.float32)
    pad = e_pad - e
    pad_idx = jnp.arange(pad, dtype=jnp.int32) % jnp.int32(n)
    src3 = jnp.concatenate([src, pad_idx]).reshape(NW, ch, CHUNK)
    dst3 = jnp.concatenate([dst, pad_idx]).reshape(NW, ch, CHUNK)
    ew3 = jnp.concatenate([ew, jnp.zeros((pad,), jnp.float32)]
                          ).reshape(NW, ch, CHUNK)
    xp = jnp.pad(x, ((0, np_ - n), (0, 0)))

    prep_k = _make_prep_kernel(np_, ch)
    spmm_k = _make_spmm_kernel(np_, ch)

    norm3 = prep_k(src3, dst3, ew3)

    rows_blk = 1280
    grid = (np_ // rows_blk,)
    vspec = pl.BlockSpec((rows_blk, F), lambda i: (i, 0))
    hspec = pl.BlockSpec((rows_blk, FH), lambda i: (i, 0))
    pspec = pl.BlockSpec((2, rows_blk, FH), lambda i: (0, i, 0))
    hshape = jax.ShapeDtypeStruct((np_, FH), jnp.float32)

    def comb_add(pa, pb):
        return pl.pallas_call(
            _comb_add_body,
            grid=grid,
            in_specs=[pspec, pspec],
            out_specs=[hspec, hspec],
            out_shape=[hshape, hshape],
        )(pa, pb)

    def comb_aff(pa, pb, preva, prevb):
        return pl.pallas_call(
            _comb_aff_body,
            grid=grid,
            in_specs=[pspec, pspec, hspec, hspec],
            out_specs=[hspec, hspec],
            out_shape=[hshape, hshape],
        )(pa, pb, preva, prevb)

    xa = xp[:, :FH]
    xb = xp[:, FH:]
    p1a, p1b = spmm_k(xa, xb, src3, dst3, norm3)
    t1a, t1b = comb_add(p1a, p1b)
    p2a, p2b = spmm_k(t1a, t1b, src3, dst3, norm3)
    t2a, t2b = comb_aff(p2a, p2b, xa, xb)
    p3a, p3b = spmm_k(t2a, t2b, src3, dst3, norm3)
    t3a, t3b = comb_aff(p3a, p3b, t1a, t1b)

    wl_pad = jnp.pad(Wlin, ((0, 0), (0, F - Wlin.shape[1])))
    bl_pad = jnp.pad(blin, (0, F - blin.shape[0])).reshape(1, F)
    wspec = pl.BlockSpec((4, F, F), lambda i: (0, 0, 0))
    bspec = pl.BlockSpec((1, F), lambda i: (0, 0))
    mspec = pl.BlockSpec((F, F), lambda i: (0, 0))

    out_pad = pl.pallas_call(
        _final_body,
        grid=grid,
        in_specs=[vspec, hspec, hspec, hspec, hspec, hspec, hspec,
                  wspec, wspec, bspec, bspec, mspec, bspec],
        out_specs=vspec,
        out_shape=jax.ShapeDtypeStruct((np_, F), jnp.float32),
    )(xp, t1a, t1b, t2a, t2b, t3a, t3b, Wxz, Wxh,
      (bxz + bhz).reshape(1, F), (bxh + bhh).reshape(1, F), wl_pad, bl_pad)

    return out_pad[:n, :1]


# speedup vs baseline: 31.9057x; 1.0478x over previous
import functools

import jax
import jax.numpy as jnp
from jax import lax
from jax.experimental import pallas as pl
from jax.experimental.pallas import tpu as pltpu
from jax.experimental.pallas import tpu_sc as plsc

NC = 2
NS = 16
NW = NC * NS
LANES = 16
CHUNK = 128
F = 128

_MESH = plsc.VectorSubcoreMesh(core_axis_name="c", subcore_axis_name="s")
_SC_PARAMS = pltpu.CompilerParams(needs_layout_passes=False,
                                  use_tc_tiling_on_sc=False)


def _rsqrt16(d):
    i = plsc.bitcast(d, jnp.int32)
    y = plsc.bitcast(jnp.int32(0x5F3759DF) - (i >> 1), jnp.float32)
    for _ in range(3):
        y = y * (1.5 - 0.5 * d * y * y)
    return jnp.where(d > 0, y, 0.0)


def _make_prep_kernel(np_, ch):
    npt = np_ // NS

    @functools.partial(
        pl.kernel,
        mesh=_MESH,
        compiler_params=_SC_PARAMS,
        out_type=jax.ShapeDtypeStruct((NW, ch, CHUNK), jnp.float32),
        scratch_types=[
            pltpu.VMEM((ch, CHUNK), jnp.int32),
            pltpu.VMEM((ch, CHUNK), jnp.float32),
            pltpu.VMEM((ch, CHUNK), jnp.int32),
            pltpu.VMEM((ch, CHUNK), jnp.float32),
            pltpu.VMEM((ch, CHUNK), jnp.int32),
            pltpu.VMEM((ch, CHUNK), jnp.float32),
            pltpu.VMEM((np_,), jnp.float32),
            pltpu.VMEM((npt,), jnp.float32),
            pltpu.VMEM_SHARED((np_,), jnp.float32),
            pltpu.SemaphoreType.DMA,
        ],
    )
    def prep_kernel(src_hbm, dst_hbm, ew_hbm, out_hbm,
                    srcA, ewA, srcB, ewB, dstv, normv, dinvv, dv, degsh,
                    sem):
        c = lax.axis_index("c")
        s = lax.axis_index("s")
        wid = c * NS + s
        pltpu.sync_copy(src_hbm.at[s], srcA)
        pltpu.sync_copy(ew_hbm.at[s], ewA)
        pltpu.sync_copy(src_hbm.at[NS + s], srcB)
        pltpu.sync_copy(ew_hbm.at[NS + s], ewB)
        pltpu.sync_copy(dst_hbm.at[wid], dstv)

        def zloop(i, carry):
            dv[pl.ds(i * LANES, LANES)] = jnp.zeros((LANES,), jnp.float32)
            return carry

        lax.fori_loop(0, npt // LANES, zloop, 0)
        pltpu.sync_copy(dv, degsh.at[pl.ds(s * npt, npt)])
        plsc.subcore_barrier()

        def deg_loop(j, carry):
            pltpu.async_copy(ewA.at[j], degsh.at[srcA.at[j]], sem,
                             add=True)
            pltpu.async_copy(ewB.at[j], degsh.at[srcB.at[j]], sem,
                             add=True)
            return carry

        lax.fori_loop(0, ch, deg_loop, 0)

        def deg_drain(j, carry):
            pltpu.make_async_copy(ewA.at[j], degsh.at[srcA.at[j]],
                                  sem).wait()
            pltpu.make_async_copy(ewB.at[j], degsh.at[srcB.at[j]],
                                  sem).wait()
            return carry

        lax.fori_loop(0, ch, deg_drain, 0)
        plsc.subcore_barrier()

        pltpu.sync_copy(degsh.at[pl.ds(s * npt, npt)], dv)

        def rloop(i, carry):
            sl = pl.ds(i * LANES, LANES)
            dv[sl] = _rsqrt16(dv[sl])
            return carry

        lax.fori_loop(0, npt // LANES, rloop, 0)
        pltpu.sync_copy(dv, degsh.at[pl.ds(s * npt, npt)])
        plsc.subcore_barrier()
        pltpu.sync_copy(degsh, dinvv)

        def jloop(j, carry, srcv, eww):
            for v in range(CHUNK // LANES):
                sl = pl.ds(v * LANES, LANES)
                i16 = srcv[j, sl]
                d16 = dstv[j, sl]
                w16 = eww[j, sl]
                gs = plsc.load_gather(dinvv, [i16])
                gd = plsc.load_gather(dinvv, [d16])
                normv[j, sl] = -(gs * w16 * gd)
            return carry

        @pl.when(c == 0)
        def _():
            lax.fori_loop(0, ch,
                          functools.partial(jloop, srcv=srcA, eww=ewA), 0)

        @pl.when(c != 0)
        def _():
            lax.fori_loop(0, ch,
                          functools.partial(jloop, srcv=srcB, eww=ewB), 0)

        pltpu.sync_copy(normv, out_hbm.at[wid])

    return prep_kernel


NBUF = 2
NIB = 2 * NBUF
FH = 64


def _make_chain_kernel(np_, ch2):
    npt = np_ // NS
    oshape = jax.ShapeDtypeStruct((NC, np_, FH), jnp.float32)

    @functools.partial(
        pl.kernel,
        mesh=_MESH,
        compiler_params=_SC_PARAMS,
        out_type=(oshape, oshape, oshape),
        scratch_types=[
            pltpu.VMEM_SHARED((np_, FH), jnp.float32),
            pltpu.VMEM_SHARED((np_, FH), jnp.float32),
            [pltpu.VMEM((CHUNK, FH), jnp.float32) for _ in range(NBUF)],
            [pltpu.VMEM((CHUNK, FH), jnp.float32) for _ in range(NBUF)],
            [pltpu.VMEM((CHUNK,), jnp.float32) for _ in range(NBUF)],
            [pltpu.VMEM((CHUNK,), jnp.int32) for _ in range(NIB)],
            [pltpu.VMEM((CHUNK,), jnp.int32) for _ in range(NIB)],
            pltpu.VMEM((CHUNK,), jnp.int32),
            [pltpu.SemaphoreType.DMA for _ in range(NBUF)],
            [pltpu.SemaphoreType.DMA for _ in range(NBUF)],
            [pltpu.SemaphoreType.DMA for _ in range(NBUF)],
            [pltpu.SemaphoreType.DMA for _ in range(NIB)],
            [pltpu.SemaphoreType.DMA for _ in range(NIB)],
        ],
    )
    def chain_kernel(xa_hbm, xb_hbm, src_hbm, dst_hbm, nrm_hbm,
                     o1_hbm, o2_hbm, o3_hbm,
                     B0, B1, grows, srows, nbufs, sslot, dslot, zidx,
                     gsem, ssem, nsem, sisem, disem):
        c = lax.axis_index("c")
        s = lax.axis_index("s")
        sl = pl.ds(s * npt, npt)
        for v in range(CHUNK // LANES):
            zidx[pl.ds(v * LANES, LANES)] = jnp.zeros((LANES,), jnp.int32)

        @pl.when(c == 0)
        def _():
            pltpu.sync_copy(xa_hbm.at[sl], B0.at[sl])

        @pl.when(c != 0)
        def _():
            pltpu.sync_copy(xb_hbm.at[sl], B0.at[sl])

        def zero_shared(buf):
            def zloop(i, carry):
                for b in range(NBUF):
                    for v in range(FH // LANES):
                        srows[b][i, pl.ds(v * LANES, LANES)] = jnp.zeros(
                            (LANES,), jnp.float32)
                return carry

            lax.fori_loop(0, CHUNK, zloop, 0)
            for k in range(npt // CHUNK):
                pltpu.sync_copy(srows[0],
                                buf.at[pl.ds(s * npt + k * CHUNK, CHUNK)])

        def spmm(srcbuf, accbuf):
            for b in range(NBUF):
                pltpu.sync_copy(src_hbm.at[s, b], sslot[b])
                pltpu.async_copy(srcbuf.at[sslot[b]], grows[b], gsem[b])
                pltpu.async_copy(nrm_hbm.at[s, b], nbufs[b], nsem[b])
                pltpu.async_copy(dst_hbm.at[s, b], dslot[b], disem[b])
                pltpu.async_copy(srows[b], accbuf.at[zidx], ssem[b],
                                 add=True)
            for q in range(NBUF, NIB):
                pltpu.async_copy(src_hbm.at[s, q], sslot[q], sisem[q])

            def round_loop(jj, carry):
                for q in range(NIB):
                    j = jj * NIB + q
                    b = q % NBUF
                    qn = (q + NBUF) % NIB
                    pltpu.make_async_copy(srcbuf.at[sslot[q]], grows[b],
                                          gsem[b]).wait()
                    pltpu.make_async_copy(nrm_hbm.at[s, 0], nbufs[b],
                                          nsem[b]).wait()
                    pltpu.make_async_copy(srows[b], accbuf.at[zidx],
                                          ssem[b]).wait()

                    def scale_loop(g, inner, b=b):
                        nv = nbufs[b][pl.ds(g * LANES, LANES)]
                        base = g * LANES
                        for r in range(LANES):
                            sc = nv[r]
                            for v in range(FH // LANES):
                                slv = pl.ds(v * LANES, LANES)
                                srows[b][base + r, slv] = (
                                    grows[b][base + r, slv] * sc)
                        return inner

                    lax.fori_loop(0, CHUNK // LANES, scale_loop, 0)
                    pltpu.make_async_copy(dst_hbm.at[s, 0], dslot[q],
                                          disem[q]).wait()
                    pltpu.async_copy(srows[b], accbuf.at[dslot[q]],
                                     ssem[b], add=True)
                    pj = jnp.minimum(j + NBUF, ch2 - 1)
                    pltpu.async_copy(dst_hbm.at[s, pj], dslot[qn],
                                     disem[qn])
                    pltpu.make_async_copy(src_hbm.at[s, 0], sslot[qn],
                                          sisem[qn]).wait()
                    pltpu.async_copy(srcbuf.at[sslot[qn]], grows[b],
                                     gsem[b])
                    pfar = jnp.minimum(j + NIB, ch2 - 1)
                    pltpu.async_copy(src_hbm.at[s, pfar], sslot[q],
                                     sisem[q])
                    pltpu.async_copy(nrm_hbm.at[s, pj], nbufs[b], nsem[b])
                return carry

            lax.fori_loop(0, ch2 // NIB, round_loop, 0)
            for b in range(NBUF):
                pltpu.make_async_copy(srcbuf.at[sslot[0]], grows[b],
                                      gsem[b]).wait()
                pltpu.make_async_copy(nrm_hbm.at[s, 0], nbufs[b],
                                      nsem[b]).wait()
                pltpu.make_async_copy(srows[b], accbuf.at[zidx],
                                      ssem[b]).wait()
                pltpu.make_async_copy(dst_hbm.at[s, 0], dslot[b],
                                      disem[b]).wait()
            for q in range(NBUF, NIB):
                pltpu.make_async_copy(src_hbm.at[s, 0], sslot[q],
                                      sisem[q]).wait()

        def combine(accbuf, load_prev, outbuf, o_hbm):
            for k in range(npt // CHUNK):
                ksl = pl.ds(s * npt + k * CHUNK, CHUNK)
                pltpu.sync_copy(accbuf.at[ksl], grows[0])
                load_prev(ksl)

                def cloop(i, carry):
                    for v in range(FH // LANES):
                        slv = pl.ds(v * LANES, LANES)
                        srows[1][i, slv] = (2.0 * grows[0][i, slv]
                                            - grows[1][i, slv])
                    return carry

                lax.fori_loop(0, CHUNK, cloop, 0)
                if outbuf is not None:
                    pltpu.sync_copy(srows[1], outbuf.at[ksl])
                pltpu.sync_copy(srows[1], o_hbm.at[c, ksl])

        def load_x(ksl):
            @pl.when(c == 0)
            def _():
                pltpu.sync_copy(xa_hbm.at[ksl], grows[1])

            @pl.when(c != 0)
            def _():
                pltpu.sync_copy(xb_hbm.at[ksl], grows[1])

        def load_t1(ksl):
            pltpu.sync_copy(o1_hbm.at[c, ksl], grows[1])

        zero_shared(B1)
        plsc.subcore_barrier()
        spmm(B0, B1)
        plsc.subcore_barrier()
        for k in range(npt // CHUNK):
            ksl = pl.ds(s * npt + k * CHUNK, CHUNK)
            pltpu.sync_copy(B1.at[ksl], o1_hbm.at[c, ksl])
        zero_shared(B0)
        plsc.subcore_barrier()
        spmm(B1, B0)
        plsc.subcore_barrier()
        combine(B0, load_x, B0, o2_hbm)
        zero_shared(B1)
        plsc.subcore_barrier()
        spmm(B0, B1)
        plsc.subcore_barrier()
        combine(B1, load_t1, None, o3_hbm)

    return chain_kernel


def _final_body(t0_ref, t1a_ref, t1b_ref, t2a_ref, t2b_ref, t3a_ref,
                t3b_ref, wz_ref, wh_ref, bz_ref, bh_ref, wl_ref, bl_ref,
                o_ref):
    x0 = t0_ref[...]
    x1 = jnp.concatenate([t1a_ref[0], t1b_ref[0]], axis=1)
    x2 = jnp.concatenate([t2a_ref[0], t2b_ref[0]], axis=1)
    x3 = jnp.concatenate([t3a_ref[0], t3b_ref[0]], axis=1)

    def conv(w_ref, b_ref):
        acc = jnp.dot(x0, w_ref[0], preferred_element_type=jnp.float32)
        acc += jnp.dot(x1, w_ref[1], preferred_element_type=jnp.float32)
        acc += jnp.dot(x2, w_ref[2], preferred_element_type=jnp.float32)
        acc += jnp.dot(x3, w_ref[3], preferred_element_type=jnp.float32)
        return acc + b_ref[...]

    az = conv(wz_ref, bz_ref)
    ah = conv(wh_ref, bh_ref)
    z = jax.nn.sigmoid(az)
    ht = jnp.tanh(ah)
    y = (1.0 - z) * ht
    o_ref[...] = (jnp.dot(y, wl_ref[...], preferred_element_type=jnp.float32)
                  + bl_ref[...])


def kernel(x, edge_index, edge_weight, Wxz, bxz, Whz, bhz, Wxr, bxr, Whr,
           bhr, Wxh, bxh, Whh, bhh, Wlin, blin):
    n, f = x.shape
    e = edge_weight.shape[0]
    np_ = ((n + 2047) // 2048) * 2048
    ch = (e + NW * CHUNK - 1) // (NW * CHUNK)
    ch = ((ch + NBUF - 1) // NBUF) * NBUF
    e_pad = NW * ch * CHUNK

    src = edge_index[0].astype(jnp.int32)
    dst = edge_index[1].astype(jnp.int32)
    ew = edge_weight.astype(jnp.float32)
    pad = e_pad - e
    pad_idx = jnp.arange(pad, dtype=jnp.int32) % jnp.int32(n)
    src3 = jnp.concatenate([src, pad_idx]).reshape(NW, ch, CHUNK)
    dst3 = jnp.concatenate([dst, pad_idx]).reshape(NW, ch, CHUNK)
    ew3 = jnp.concatenate([ew, jnp.zeros((pad,), jnp.float32)]
                          ).reshape(NW, ch, CHUNK)
    xp = jnp.pad(x, ((0, np_ - n), (0, 0)))

    prep_k = _make_prep_kernel(np_, ch)
    chain_k = _make_chain_kernel(np_, 2 * ch)

    norm3 = prep_k(src3, dst3, ew3)

    ch2 = 2 * ch
    src2 = src3.reshape(NS, ch2, CHUNK)
    dst2 = dst3.reshape(NS, ch2, CHUNK)
    norm2 = norm3.reshape(NS, ch2, CHUNK)

    xa = xp[:, :FH]
    xb = xp[:, FH:]
    t1, t2, t3 = chain_k(xa, xb, src2, dst2, norm2)

    rows_blk = 1280
    grid = (np_ // rows_blk,)
    vspec = pl.BlockSpec((rows_blk, F), lambda i: (i, 0))
    haspec = pl.BlockSpec((1, rows_blk, FH), lambda i: (0, i, 0))
    hbspec = pl.BlockSpec((1, rows_blk, FH), lambda i: (1, i, 0))

    wl_pad = jnp.pad(Wlin, ((0, 0), (0, F - Wlin.shape[1])))
    bl_pad = jnp.pad(blin, (0, F - blin.shape[0])).reshape(1, F)
    wspec = pl.BlockSpec((4, F, F), lambda i: (0, 0, 0))
    bspec = pl.BlockSpec((1, F), lambda i: (0, 0))
    mspec = pl.BlockSpec((F, F), lambda i: (0, 0))

    out_pad = pl.pallas_call(
        _final_body,
        grid=grid,
        in_specs=[vspec, haspec, hbspec, haspec, hbspec, haspec, hbspec,
                  wspec, wspec, bspec, bspec, mspec, bspec],
        out_specs=vspec,
        out_shape=jax.ShapeDtypeStruct((np_, F), jnp.float32),
    )(xp, t1, t1, t2, t2, t3, t3, Wxz, Wxh,
      (bxz + bhz).reshape(1, F), (bxh + bhh).reshape(1, F), wl_pad, bl_pad)

    return out_pad[:n, :1]
